# baseline (speedup 1.0000x reference)
import functools

import jax
import jax.numpy as jnp
from jax import lax
from jax.experimental import pallas as pl
from jax.experimental.pallas import tpu as pltpu
from jax.experimental.pallas import tpu_sc as plsc

HIDDEN = 1024
ROWS = 4 * 8192
NC, NS = 2, 16
NW = NC * NS
RPW = ROWS // NW
CH = 64
NCHUNK = RPW // CH

_mesh = plsc.VectorSubcoreMesh(core_axis_name="c", subcore_axis_name="s")


@functools.partial(
    pl.kernel,
    mesh=_mesh,
    out_type=jax.ShapeDtypeStruct((ROWS, HIDDEN), jnp.float32),
    scratch_types=[
        pltpu.VMEM((RPW,), jnp.int32),
        pltpu.VMEM((CH, HIDDEN), jnp.float32),
        pltpu.SemaphoreType.DMA,
    ],
)
def _sc_lookup(seg_hbm, table_hbm, out_hbm, idx_v, buf, gsem):
    wid = lax.axis_index("s") * NC + lax.axis_index("c")
    base = wid * RPW
    pltpu.sync_copy(seg_hbm.at[pl.ds(base, RPW)], idx_v)

    def body(c, carry):
        off = pl.multiple_of(c * CH, CH)
        pltpu.async_copy(
            table_hbm.at[idx_v.at[pl.ds(off, CH)]], buf, gsem
        ).wait()
        pltpu.sync_copy(buf, out_hbm.at[pl.ds(base + off, CH)])
        return carry

    lax.fori_loop(0, NCHUNK, body, 0)


def kernel(segments, table):
    seg = segments.reshape(ROWS).astype(jnp.int32)
    out = _sc_lookup(seg, table)
    return out.reshape(segments.shape[0], segments.shape[1], HIDDEN)


# speedup vs baseline: 3.9406x; 3.9406x over previous
import functools

import jax
import jax.numpy as jnp
from jax import lax
from jax.experimental import pallas as pl
from jax.experimental.pallas import tpu as pltpu
from jax.experimental.pallas import tpu_sc as plsc

HIDDEN = 1024
ROWS = 4 * 8192
NC, NS = 2, 16
NW = NC * NS
RPW = ROWS // NW
GR = 16
GROUPS = RPW // GR
NBUF = 4
JCH = HIDDEN // 16
GSZ = GR * HIDDEN

_mesh = plsc.VectorSubcoreMesh(core_axis_name="c", subcore_axis_name="s")

_DIMS = lax.GatherDimensionNumbers(
    offset_dims=(), collapsed_slice_dims=(0,), start_index_map=(0,)
)


def _lane_splat(vec, lane):
    return lax.gather(
        vec,
        jnp.full((16, 1), lane, jnp.int32),
        _DIMS,
        (1,),
        mode=lax.GatherScatterMode.PROMISE_IN_BOUNDS,
    )


@functools.partial(
    pl.kernel,
    mesh=_mesh,
    out_type=jax.ShapeDtypeStruct((ROWS * HIDDEN,), jnp.float32),
    scratch_types=[
        pltpu.VMEM((RPW,), jnp.int32),
        pltpu.VMEM((2 * HIDDEN,), jnp.float32),
        pltpu.VMEM((NBUF * GSZ,), jnp.float32),
        pltpu.SemaphoreType.DMA,
    ],
)
def _sc_lookup(seg_hbm, table_hbm, out_hbm, idx_v, tab_v, buf, ssem):
    wid = lax.axis_index("s") * NC + lax.axis_index("c")
    base = wid * RPW
    pltpu.sync_copy(seg_hbm.at[pl.ds(base, RPW)], idx_v)
    pltpu.sync_copy(table_hbm, tab_v)

    def wait_one_scatter():
        pltpu.make_async_copy(
            out_hbm.at[pl.ds(0, GSZ)], buf.at[pl.ds(0, GSZ)], ssem
        ).wait()

    def outer(o, carry):
        for b in range(NBUF):
            g = o * NBUF + b
            off = pl.multiple_of(g * GR, GR)
            idx16 = idx_v[pl.ds(off, 16)]
            mults = [
                _lane_splat(idx16, r).astype(jnp.float32) for r in range(GR)
            ]

            @pl.when(o > 0)
            def _():
                wait_one_scatter()

            def jbody(j, c, _b=b, _mults=mults):
                jo = pl.multiple_of(j * 16, 16)
                t0 = tab_v[pl.ds(jo, 16)]
                d = tab_v[pl.ds(HIDDEN + jo, 16)] - t0
                for r in range(GR):
                    buf[pl.ds(_b * GSZ + r * HIDDEN + jo, 16)] = (
                        t0 + _mults[r] * d
                    )
                return c

            lax.fori_loop(0, JCH, jbody, 0)
            pltpu.async_copy(
                buf.at[pl.ds(b * GSZ, GSZ)],
                out_hbm.at[pl.ds((base + off) * HIDDEN, GSZ)],
                ssem,
            )
        return carry

    lax.fori_loop(0, GROUPS // NBUF, outer, 0)
    for _ in range(NBUF):
        wait_one_scatter()


def kernel(segments, table):
    seg = segments.reshape(ROWS).astype(jnp.int32)
    out = _sc_lookup(seg, table.reshape(2 * HIDDEN))
    return out.reshape(segments.shape[0], segments.shape[1], HIDDEN)


# speedup vs baseline: 9.9602x; 2.5276x over previous
import functools

import jax
import jax.numpy as jnp
from jax import lax
from jax.experimental import pallas as pl
from jax.experimental.pallas import tpu as pltpu
from jax.experimental.pallas import tpu_sc as plsc

HIDDEN = 1024
ROWS = 4 * 8192
NC, NS = 2, 16
NW = NC * NS
RPW = ROWS // NW
GR = 16
GROUPS = RPW // GR
NBUF = 4
JCH = HIDDEN // 16

_mesh = plsc.VectorSubcoreMesh(core_axis_name="c", subcore_axis_name="s")

_DIMS = lax.GatherDimensionNumbers(
    offset_dims=(), collapsed_slice_dims=(0,), start_index_map=(0,)
)


def _lane_splat(vec, lane):
    return lax.gather(
        vec,
        jnp.full((16, 1), lane, jnp.int32),
        _DIMS,
        (1,),
        mode=lax.GatherScatterMode.PROMISE_IN_BOUNDS,
    )


@functools.partial(
    pl.kernel,
    mesh=_mesh,
    out_type=jax.ShapeDtypeStruct((ROWS, HIDDEN), jnp.float32),
    scratch_types=[
        pltpu.VMEM((RPW,), jnp.int32),
        pltpu.VMEM((2 * HIDDEN,), jnp.float32),
        pltpu.VMEM((NBUF, GR, HIDDEN), jnp.float32),
        pltpu.SemaphoreType.DMA,
    ],
)
def _sc_lookup(seg_hbm, table_hbm, out_hbm, idx_v, tab_v, bufs, ssem):
    wid = lax.axis_index("s") * NC + lax.axis_index("c")
    base = wid * RPW
    pltpu.sync_copy(seg_hbm.at[pl.ds(base, RPW)], idx_v)
    pltpu.sync_copy(table_hbm, tab_v)

    def wait_one_scatter():
        pltpu.make_async_copy(
            out_hbm.at[pl.ds(base, GR)], bufs.at[0], ssem
        ).wait()

    def outer(o, carry):
        for b in range(NBUF):
            g = o * NBUF + b
            off = pl.multiple_of(g * GR, GR)
            idx16 = idx_v[pl.ds(off, 16)]
            mults = [
                _lane_splat(idx16, r).astype(jnp.float32) for r in range(GR)
            ]

            @pl.when(o > 0)
            def _():
                wait_one_scatter()

            def jbody(j, c, _b=b, _mults=mults):
                jo = pl.multiple_of(j * 16, 16)
                t0 = tab_v[pl.ds(jo, 16)]
                d = tab_v[pl.ds(HIDDEN + jo, 16)] - t0
                for r in range(GR):
                    bufs.at[_b].at[r][pl.ds(jo, 16)] = t0 + _mults[r] * d
                return c

            lax.fori_loop(0, JCH, jbody, 0)
            pltpu.async_copy(
                bufs.at[b], out_hbm.at[pl.ds(base + off, GR)], ssem
            )
        return carry

    lax.fori_loop(0, GROUPS // NBUF, outer, 0)
    for _ in range(NBUF):
        wait_one_scatter()


def kernel(segments, table):
    seg = segments.reshape(ROWS).astype(jnp.int32)
    out = _sc_lookup(seg, table.reshape(2 * HIDDEN))
    return out.reshape(segments.shape[0], segments.shape[1], HIDDEN)


# speedup vs baseline: 10.0243x; 1.0064x over previous
import functools

import jax
import jax.numpy as jnp
from jax import lax
from jax.experimental import pallas as pl
from jax.experimental.pallas import tpu as pltpu
from jax.experimental.pallas import tpu_sc as plsc

HIDDEN = 1024
BATCH = 4
SEQ = 8192
ROWS = BATCH * SEQ
NC, NS = 2, 16
NW = NC * NS
RPW = ROWS // NW
WPB = SEQ // RPW
GR = 16
GROUPS = RPW // GR
NBUF = 2
JCH = HIDDEN // 16

_mesh = plsc.VectorSubcoreMesh(core_axis_name="c", subcore_axis_name="s")

_DIMS = lax.GatherDimensionNumbers(
    offset_dims=(), collapsed_slice_dims=(0,), start_index_map=(0,)
)


def _lane_splat(vec, lane):
    return lax.gather(
        vec,
        jnp.full((16, 1), lane, jnp.int32),
        _DIMS,
        (1,),
        mode=lax.GatherScatterMode.PROMISE_IN_BOUNDS,
    )


@functools.partial(
    pl.kernel,
    mesh=_mesh,
    out_type=jax.ShapeDtypeStruct((BATCH, SEQ, HIDDEN), jnp.float32),
    scratch_types=[
        pltpu.VMEM((RPW,), jnp.int32),
        pltpu.VMEM((2, HIDDEN), jnp.float32),
        pltpu.VMEM((NBUF, GR, HIDDEN), jnp.float32),
        pltpu.SemaphoreType.DMA,
    ],
)
def _sc_lookup(seg_hbm, table_hbm, out_hbm, idx_v, tab_v, bufs, ssem):
    wid = lax.axis_index("s") * NC + lax.axis_index("c")
    bi = lax.div(wid, WPB)
    srow = lax.rem(wid, WPB) * RPW
    pltpu.sync_copy(seg_hbm.at[bi].at[pl.ds(srow, RPW)], idx_v)
    pltpu.sync_copy(table_hbm, tab_v)
    out_w = out_hbm.at[bi]

    def wait_one_scatter():
        pltpu.make_async_copy(
            out_w.at[pl.ds(srow, GR)], bufs.at[0], ssem
        ).wait()

    def outer(o, carry):
        for b in range(NBUF):
            g = o * NBUF + b
            off = pl.multiple_of(g * GR, GR)
            idx16 = idx_v[pl.ds(off, 16)]
            mults = [
                _lane_splat(idx16, r).astype(jnp.float32) for r in range(GR)
            ]

            @pl.when(o > 0)
            def _():
                wait_one_scatter()

            def jbody(j, c, _b=b, _mults=mults):
                jo = pl.multiple_of(j * 16, 16)
                t0 = tab_v.at[0][pl.ds(jo, 16)]
                d = tab_v.at[1][pl.ds(jo, 16)] - t0
                for r in range(GR):
                    bufs.at[_b].at[r][pl.ds(jo, 16)] = t0 + _mults[r] * d
                return c

            lax.fori_loop(0, JCH, jbody, 0)
            pltpu.async_copy(
                bufs.at[b], out_w.at[pl.ds(srow + off, GR)], ssem
            )
        return carry

    lax.fori_loop(0, GROUPS // NBUF, outer, 0)
    for _ in range(NBUF):
        wait_one_scatter()


def kernel(segments, table):
    return _sc_lookup(segments.astype(jnp.int32), table)
